# baseline (speedup 1.0000x reference)
import functools

import numpy as np
import jax
import jax.numpy as jnp
from jax import lax
from jax.experimental import pallas as pl
from jax.experimental.pallas import tpu as pltpu
from jax.experimental.pallas import tpu_sc as plsc

_VOCAB = 1000000
_D = 64
_W = 2048
_B = 4

_NC = 2
_NS = 16
_NW = _NC * _NS
_BTOT = _B * _W
_BPW = _BTOT // _NW
_WPR = _W // _BPW
_SCALE = float(np.sqrt(_D))
_LANES = 128
_RING = 8


def _pos_encoding_t() -> np.ndarray:
    half = _D // 2
    positions = np.arange(_W, dtype=np.float32)[:, None]
    depths = np.arange(half, dtype=np.float32)[None, :] / float(half)
    angle_rads = positions * (1.0 / np.power(10000.0, depths))
    pe = np.concatenate([np.sin(angle_rads), np.cos(angle_rads)], axis=-1)
    return np.ascontiguousarray(pe.T.astype(np.float32))


_PET = _pos_encoding_t()


@functools.partial(
    pl.kernel,
    mesh=plsc.VectorSubcoreMesh(core_axis_name="c", subcore_axis_name="s"),
    out_type=jax.ShapeDtypeStruct((_B, _D, _W), jnp.float32),
    scratch_types=[
        pltpu.VMEM((_BPW + 16,), jnp.int32),
        pltpu.VMEM((_BPW, _D), jnp.float32),
        pltpu.VMEM((_D, _BPW), jnp.float32),
        pltpu.VMEM((_RING, _D, _LANES), jnp.float32),
        [pltpu.SemaphoreType.DMA] * _RING,
    ],
    compiler_params=pltpu.CompilerParams(needs_layout_passes=False),
)
def _embed_pe(x_hbm, tablet_hbm, pet_hbm, out_hbm, idx_v, rows_v, cols_v, tbuf, sems):
    wid = lax.axis_index("s") * _NC + lax.axis_index("c")
    base = wid * _BPW
    part = lax.rem(wid, _WPR)

    pltpu.sync_copy(x_hbm.at[pl.ds(base, _BPW)], idx_v.at[pl.ds(0, _BPW)])

    iota16 = lax.iota(jnp.int32, 16)
    scale = jnp.float32(_SCALE)

    def fire(r, fvec, lane):
        v = fvec[lane]
        col = pl.multiple_of(v - lax.rem(v, _LANES), _LANES)
        pltpu.async_copy(tablet_hbm.at[:, pl.ds(col, _LANES)], tbuf.at[r], sems[r])

    vec0 = idx_v[pl.ds(0, 16)]
    for r in range(_RING):
        fire(r, vec0, r)

    pltpu.sync_copy(pet_hbm.at[:, pl.ds(part * _BPW, _BPW)], cols_v)

    def body(s, carry):
        vec = idx_v[pl.ds(s * 16, 16)]
        vec_next = idx_v[pl.ds(s * 16 + 16, 16)]
        for k in range(16):
            r = k % _RING
            i = s * 16 + k
            pltpu.make_async_copy(
                tablet_hbm.at[:, pl.ds(0, _LANES)], tbuf.at[r], sems[r]
            ).wait()
            v = vec[k]
            lvec = jnp.full((16,), lax.rem(v, _LANES), jnp.int32)
            for g in range(4):
                sl = pl.ds(g * 16, 16)
                rows_v[i, sl] = plsc.load_gather(tbuf.at[r], [iota16 + g * 16, lvec])

            fvec = vec if k < 16 - _RING else vec_next
            lane = (k + _RING) % 16

            @pl.when(i + _RING < _BPW)
            def _():
                fire(r, fvec, lane)

        return carry

    lax.fori_loop(0, _BPW // 16, body, 0, unroll=False)

    def tr_body(d, carry):
        dvec = jnp.full((16,), d, jnp.int32)
        for m in range(_BPW // 16):
            sl = pl.ds(m * 16, 16)
            vals = plsc.load_gather(rows_v, [iota16 + m * 16, dvec])
            cols_v[d, sl] = vals * scale + cols_v[d, sl]
        return carry

    lax.fori_loop(0, _D, tr_body, 0, unroll=False)

    b = wid // _WPR
    pltpu.sync_copy(cols_v, out_hbm.at[b, :, pl.ds(part * _BPW, _BPW)])


def kernel(x, table):
    pet = jnp.asarray(_PET)
    idx = x.reshape(_BTOT).astype(jnp.int32)
    out = _embed_pe(idx, table.T, pet)
    return out.transpose(0, 2, 1)


# speedup vs baseline: 1.1148x; 1.1148x over previous
import functools

import numpy as np
import jax
import jax.numpy as jnp
from jax import lax
from jax.experimental import pallas as pl
from jax.experimental.pallas import tpu as pltpu
from jax.experimental.pallas import tpu_sc as plsc

_VOCAB = 1000000
_D = 64
_W = 2048
_B = 4

_NC = 2
_NS = 16
_NW = _NC * _NS
_BTOT = _B * _W
_BPW = _BTOT // _NW
_WPR = _W // _BPW
_SCALE = float(np.sqrt(_D))
_LANES = 128
_RING = 8


def _pos_encoding_t() -> np.ndarray:
    half = _D // 2
    positions = np.arange(_W, dtype=np.float32)[:, None]
    depths = np.arange(half, dtype=np.float32)[None, :] / float(half)
    angle_rads = positions * (1.0 / np.power(10000.0, depths))
    pe = np.concatenate([np.sin(angle_rads), np.cos(angle_rads)], axis=-1)
    return np.ascontiguousarray(pe.T.astype(np.float32))


_PET = _pos_encoding_t()


@functools.partial(
    pl.kernel,
    mesh=plsc.VectorSubcoreMesh(core_axis_name="c", subcore_axis_name="s"),
    out_type=jax.ShapeDtypeStruct((_B, _D, _W), jnp.float32),
    scratch_types=[
        pltpu.VMEM((_BPW + 16,), jnp.int32),
        pltpu.VMEM((_D, _BPW), jnp.float32),
        pltpu.VMEM((_RING, _D, _LANES), jnp.float32),
        [pltpu.SemaphoreType.DMA] * _RING,
    ],
    compiler_params=pltpu.CompilerParams(needs_layout_passes=False),
)
def _embed_pe(x_hbm, tablet_hbm, pet_hbm, out_hbm, idx_v, cols_v, tbuf, sems):
    wid = lax.axis_index("s") * _NC + lax.axis_index("c")
    base = wid * _BPW
    part = lax.rem(wid, _WPR)

    pltpu.sync_copy(x_hbm.at[pl.ds(base, _BPW)], idx_v.at[pl.ds(0, _BPW)])

    iota16 = lax.iota(jnp.int32, 16)
    scale = jnp.float32(_SCALE)

    def fire(r, fvec, lane):
        v = fvec[lane]
        col = pl.multiple_of(v - lax.rem(v, _LANES), _LANES)
        pltpu.async_copy(tablet_hbm.at[:, pl.ds(col, _LANES)], tbuf.at[r], sems[r])

    vec0 = idx_v[pl.ds(0, 16)]
    for r in range(_RING):
        fire(r, vec0, r)

    pltpu.sync_copy(pet_hbm.at[:, pl.ds(part * _BPW, _BPW)], cols_v)

    def body(s, carry):
        vec = idx_v[pl.ds(s * 16, 16)]
        vec_next = idx_v[pl.ds(s * 16 + 16, 16)]
        for k in range(16):
            r = k % _RING
            i = s * 16 + k
            pltpu.make_async_copy(
                tablet_hbm.at[:, pl.ds(0, _LANES)], tbuf.at[r], sems[r]
            ).wait()
            v = vec[k]
            lvec = jnp.full((16,), lax.rem(v, _LANES), jnp.int32)
            ivec = jnp.full((16,), i, jnp.int32)
            for g in range(4):
                vals = plsc.load_gather(tbuf.at[r], [iota16 + g * 16, lvec])
                plsc.addupdate_scatter(
                    cols_v, [iota16 + g * 16, ivec], vals * scale
                )

            fvec = vec if k < 16 - _RING else vec_next
            lane = (k + _RING) % 16

            @pl.when(i + _RING < _BPW)
            def _():
                fire(r, fvec, lane)

        return carry

    lax.fori_loop(0, _BPW // 16, body, 0, unroll=False)

    b = wid // _WPR
    pltpu.sync_copy(cols_v, out_hbm.at[b, :, pl.ds(part * _BPW, _BPW)])


def kernel(x, table):
    pet = jnp.asarray(_PET)
    idx = x.reshape(_BTOT).astype(jnp.int32)
    out = _embed_pe(idx, table.T, pet)
    return out.transpose(0, 2, 1)
